# Initial kernel scaffold; baseline (speedup 1.0000x reference)
import jax
import jax.numpy as jnp
from jax.experimental import pallas as pl


def kernel(query, reference_points, value_flat, spatial_shapes, level_start_index, W_off, b_off, W_attn, b_attn, W_val, b_val, W_out, b_out):
    raise NotImplementedError("write your pallas kernel here")


# speedup vs baseline: 63.3562x; 63.3562x over previous
import functools

import jax
import jax.numpy as jnp
from jax import lax
from jax.experimental import pallas as pl
from jax.experimental.pallas import tpu as pltpu
from jax.experimental.pallas import tpu_sc as plsc

H_HEADS = 8
N_LVL = 3
N_PTS = 4
D_MODEL = 256
HEAD_DIM = 32
HLP = H_HEADS * N_LVL * N_PTS

_B = 2
_LQ = 13125
_NQ = _B * _LQ

_NW = 32
_QW = 824
_NQ_PAD = _QW * _NW
_ROW_BLK = 824
_GRID = _NQ_PAD // _ROW_BLK
_G = 8
_TERMS = _G * 4 * HLP
_IDX_ROWS = _TERMS // 128


def _col_consts(dtype=jnp.float32):
    col = lax.broadcasted_iota(jnp.int32, (1, HLP), 1)
    lc = (col % (N_LVL * N_PTS)) // N_PTS
    hc = col // (N_LVL * N_PTS)
    wf = jnp.where(lc == 0, 100.0, jnp.where(lc == 1, 50.0, 25.0)).astype(dtype)
    lsi = jnp.where(lc == 0, 0.0, jnp.where(lc == 1, 10000.0, 12500.0)).astype(dtype)
    return wf, lsi, hc.astype(dtype)


def _prep_body(q_ref, vf_ref, rp_ref, wval_ref, bval_ref, woffy_ref, woffx_ref,
               boffy_ref, boffx_ref, wattn_ref, battn_ref, val_ref, idx_ref, wgt_ref):
    i = pl.program_id(0)
    row0 = i * _ROW_BLK
    rowi = row0 + lax.broadcasted_iota(jnp.int32, (_ROW_BLK, 1), 0)
    bf = jnp.minimum(rowi // _LQ, _B - 1).astype(jnp.float32)

    hi = lax.Precision.HIGHEST
    vf = vf_ref[...]
    val_ref[...] = (
        jnp.dot(vf, wval_ref[...], preferred_element_type=jnp.float32, precision=hi)
        + bval_ref[...])

    q = q_ref[...]
    offy = jnp.dot(q, woffy_ref[...], preferred_element_type=jnp.float32, precision=hi) + boffy_ref[...]
    offx = jnp.dot(q, woffx_ref[...], preferred_element_type=jnp.float32, precision=hi) + boffx_ref[...]
    logits = jnp.dot(q, wattn_ref[...], preferred_element_type=jnp.float32, precision=hi) + battn_ref[...]

    colg = lax.broadcasted_iota(jnp.int32, (HLP, HLP), 0) // (N_LVL * N_PTS)
    colg2 = lax.broadcasted_iota(jnp.int32, (HLP, HLP), 1) // (N_LVL * N_PTS)
    grp = (colg == colg2).astype(jnp.float32)
    e = jnp.exp(logits)
    attn = e / jnp.dot(e, grp, preferred_element_type=jnp.float32, precision=hi)

    rp = rp_ref[...]
    lcb = (lax.broadcasted_iota(jnp.int32, (1, HLP), 1) % (N_LVL * N_PTS)) // N_PTS
    def _sel(col_for_lvl):
        a0 = rp[:, col_for_lvl(0):col_for_lvl(0) + 1]
        a1 = rp[:, col_for_lvl(1):col_for_lvl(1) + 1]
        a2 = rp[:, col_for_lvl(2):col_for_lvl(2) + 1]
        return jnp.where(lcb == 0, a0, jnp.where(lcb == 1, a1, a2))
    rpy = _sel(lambda l: 2 * l)
    rpx = _sel(lambda l: 2 * l + 1)

    wf, lsi, hcf = _col_consts()
    x = rpx * wf + offx - 0.5
    y = rpy * wf + offy - 0.5
    x0 = jnp.floor(x)
    y0 = jnp.floor(y)
    fx = x - x0
    fy = y - y0
    wmax = wf - 1.0
    vx0 = ((x0 >= 0.0) & (x0 <= wmax)).astype(jnp.float32)
    vx1 = ((x0 >= -1.0) & (x0 <= wmax - 1.0)).astype(jnp.float32)
    vy0 = ((y0 >= 0.0) & (y0 <= wmax)).astype(jnp.float32)
    vy1 = ((y0 >= -1.0) & (y0 <= wmax - 1.0)).astype(jnp.float32)
    cx0 = jnp.clip(x0, 0.0, wmax)
    cx1 = jnp.clip(x0 + 1.0, 0.0, wmax)
    cy0 = jnp.clip(y0, 0.0, wmax)
    cy1 = jnp.clip(y0 + 1.0, 0.0, wmax)

    base = bf * jnp.float32(_LQ) + lsi
    gx0 = 1.0 - fx
    gy0 = 1.0 - fy
    corners = (
        (cy0, cx0, gy0 * gx0 * vy0 * vx0),
        (cy0, cx1, gy0 * fx * vy0 * vx1),
        (cy1, cx0, fy * gx0 * vy1 * vx0),
        (cy1, cx1, fy * fx * vy1 * vx1),
    )
    for c, (cy, cx, wbi) in enumerate(corners):
        rowf = (base + cy * wf + cx) * jnp.float32(H_HEADS) + hcf
        idx_ref[:, c, :] = rowf.astype(jnp.int32)
        wgt_ref[:, c, :] = attn * wbi


def _prep_call(q2, vf2, rp2, W_val, b_val, W_off, b_off, W_attn, b_attn):
    woffy = W_off[:, 0::2]
    woffx = W_off[:, 1::2]
    boffy = b_off[0::2].reshape(1, HLP)
    boffx = b_off[1::2].reshape(1, HLP)
    row_spec = pl.BlockSpec((_ROW_BLK, D_MODEL), lambda i: (i, 0))
    full = lambda shape: pl.BlockSpec(shape, lambda i: tuple(0 for _ in shape))
    return pl.pallas_call(
        _prep_body,
        grid=(_GRID,),
        in_specs=[
            row_spec,
            row_spec,
            pl.BlockSpec((_ROW_BLK, 2 * N_LVL), lambda i: (i, 0)),
            full((D_MODEL, D_MODEL)),
            full((1, D_MODEL)),
            full((D_MODEL, HLP)),
            full((D_MODEL, HLP)),
            full((1, HLP)),
            full((1, HLP)),
            full((D_MODEL, HLP)),
            full((1, HLP)),
        ],
        out_specs=[
            row_spec,
            pl.BlockSpec((_ROW_BLK, 4, HLP), lambda i: (i, 0, 0)),
            pl.BlockSpec((_ROW_BLK, 4, HLP), lambda i: (i, 0, 0)),
        ],
        out_shape=[
            jax.ShapeDtypeStruct((_NQ_PAD, D_MODEL), jnp.float32),
            jax.ShapeDtypeStruct((_NQ_PAD, 4, HLP), jnp.int32),
            jax.ShapeDtypeStruct((_NQ_PAD, 4, HLP), jnp.float32),
        ],
    )(q2, vf2, rp2, W_val, b_val.reshape(1, D_MODEL), woffy, woffx,
      boffy, boffx, W_attn, b_attn.reshape(1, HLP))


def _sc_body(table_hbm, idx_hbm, w_hbm, out_hbm, idx_v, w_v, rows_v, out_v, sem):
    wid = lax.axis_index("s") * 2 + lax.axis_index("c")
    base_q = wid * _QW

    def chunk(g, carry):
        q0 = base_q + g * _G
        t0 = pl.multiple_of(q0 * (4 * HLP), 1024)
        r0 = pl.multiple_of(q0 * (4 * HLP) // 128, 8)
        pltpu.sync_copy(idx_hbm.at[pl.ds(r0, _IDX_ROWS)], idx_v)
        pltpu.sync_copy(w_hbm.at[pl.ds(t0, _TERMS)], w_v.at[pl.ds(0, _TERMS)])
        copies = []
        for k in range(_IDX_ROWS):
            copies.append(pltpu.async_copy(
                table_hbm.at[idx_v.at[k]], rows_v.at[pl.ds(k * 128, 128)], sem))
        for c in copies:
            c.wait()

        def one_out(o, carry2):
            r = o // H_HEADS
            h = o % H_HEADS
            tb = r * (4 * HLP) + h * (N_LVL * N_PTS)
            acc0 = jnp.zeros((16,), jnp.float32)
            acc1 = jnp.zeros((16,), jnp.float32)
            for c in range(4):
                w16 = w_v[pl.ds(tb + c * HLP, 16)]
                for j in range(N_LVL * N_PTS):
                    t = tb + c * HLP + j
                    wb = w16[j]
                    acc0 = acc0 + wb * rows_v[t, pl.ds(0, 16)]
                    acc1 = acc1 + wb * rows_v[t, pl.ds(16, 16)]
            out_v[o, pl.ds(0, 16)] = acc0
            out_v[o, pl.ds(16, 16)] = acc1
            return carry2

        lax.fori_loop(0, _G * H_HEADS, one_out, 0)
        o0 = pl.multiple_of(q0 * H_HEADS, 64)
        pltpu.sync_copy(out_v, out_hbm.at[pl.ds(o0, _G * H_HEADS)])
        return carry

    lax.fori_loop(0, _QW // _G, chunk, 0)


def _sc_call(table, idx_flat2d, w_flat):
    mesh = plsc.VectorSubcoreMesh(core_axis_name="c", subcore_axis_name="s",
                                  num_cores=2, num_subcores=16)
    f = pl.kernel(
        _sc_body,
        out_type=jax.ShapeDtypeStruct((_NQ_PAD * H_HEADS, HEAD_DIM), jnp.float32),
        mesh=mesh,
        scratch_types=[
            pltpu.VMEM((_IDX_ROWS, 128), jnp.int32),
            pltpu.VMEM((_TERMS + 16,), jnp.float32),
            pltpu.VMEM((_TERMS, HEAD_DIM), jnp.float32),
            pltpu.VMEM((_G * H_HEADS, HEAD_DIM), jnp.float32),
            pltpu.SemaphoreType.DMA,
        ],
        compiler_params=pltpu.CompilerParams(use_tc_tiling_on_sc=False),
    )
    return f(table, idx_flat2d, w_flat)


def _out_body(x_ref, w_ref, b_ref, o_ref):
    o_ref[...] = (
        jnp.dot(x_ref[...], w_ref[...], preferred_element_type=jnp.float32,
                precision=lax.Precision.HIGHEST)
        + b_ref[...])


def _out_call(x2, W_out, b_out):
    row_spec = pl.BlockSpec((_ROW_BLK, D_MODEL), lambda i: (i, 0))
    return pl.pallas_call(
        _out_body,
        grid=(_GRID,),
        in_specs=[
            row_spec,
            pl.BlockSpec((D_MODEL, D_MODEL), lambda i: (0, 0)),
            pl.BlockSpec((1, D_MODEL), lambda i: (0, 0)),
        ],
        out_specs=row_spec,
        out_shape=jax.ShapeDtypeStruct((_NQ_PAD, D_MODEL), jnp.float32),
    )(x2, W_out, b_out.reshape(1, D_MODEL))


def kernel(query, reference_points, value_flat, spatial_shapes, level_start_index,
           W_off, b_off, W_attn, b_attn, W_val, b_val, W_out, b_out):
    B, Lq, d_model = query.shape
    pad = _NQ_PAD - _NQ
    q2 = jnp.pad(query.reshape(_NQ, D_MODEL), ((0, pad), (0, 0)))
    vf2 = jnp.pad(value_flat.reshape(_NQ, D_MODEL), ((0, pad), (0, 0)))
    rp2 = jnp.pad(reference_points.reshape(_NQ, 2 * N_LVL), ((0, pad), (0, 0)))

    val, idx, wgt = _prep_call(q2, vf2, rp2, W_val, b_val, W_off, b_off, W_attn, b_attn)

    table = val.reshape(_NQ_PAD * H_HEADS, HEAD_DIM)
    idx2d = idx.reshape(_NQ_PAD * 4 * HLP // 128, 128)
    wflat = wgt.reshape(_NQ_PAD * 4 * HLP)

    sc_out = _sc_call(table, idx2d, wflat)

    x2 = sc_out.reshape(_NQ_PAD, D_MODEL)
    out = _out_call(x2, W_out, b_out)
    return out[:_NQ].reshape(B, Lq, d_model)


# speedup vs baseline: 81.7842x; 1.2909x over previous
import numpy as np

import jax
import jax.numpy as jnp
from jax import lax
from jax.experimental import pallas as pl
from jax.experimental.pallas import tpu as pltpu
from jax.experimental.pallas import tpu_sc as plsc

H_HEADS = 8
N_LVL = 3
N_PTS = 4
D_MODEL = 256
HEAD_DIM = 32
HLP = H_HEADS * N_LVL * N_PTS

_B = 2
_LQ = 13125
_NQ = _B * _LQ

_NW = 32
_QW = 824
_NQ_PAD = _QW * _NW
_ROW_BLK = 824
_GRID = _NQ_PAD // _ROW_BLK
_G = 8
_NCH = _QW // _G
_TERMS = _G * 4 * HLP
_IDX_ROWS = _TERMS // 128
_WORDS = HEAD_DIM // 2

_VAL_PERM = np.concatenate([
    np.arange(128).reshape(8, 16) // 16 * 32 + np.arange(16)[None, :],
    np.arange(128).reshape(8, 16) // 16 * 32 + 16 + np.arange(16)[None, :],
]).reshape(-1)


def _prep_body(q_ref, vf_ref, rp_ref, wval_ref, bval_ref, woffy_ref, woffx_ref,
               boffy_ref, boffx_ref, wattn_ref, battn_ref, val_ref, idx_ref, wgt_ref):
    i = pl.program_id(0)
    row0 = i * _ROW_BLK
    rowi = row0 + lax.broadcasted_iota(jnp.int32, (_ROW_BLK, 1), 0)
    live = rowi < _NQ
    bf = jnp.minimum(rowi // _LQ, _B - 1).astype(jnp.float32)

    hi = lax.Precision.HIGHEST
    vf = jnp.where(live, vf_ref[...], 0.0)
    val = (jnp.dot(vf, wval_ref[...], preferred_element_type=jnp.float32, precision=hi)
           + bval_ref[...])
    lo_u = lax.bitcast_convert_type(val[:, :128].astype(jnp.bfloat16),
                                    jnp.uint16).astype(jnp.uint32)
    hi_u = lax.bitcast_convert_type(val[:, 128:].astype(jnp.bfloat16),
                                    jnp.uint16).astype(jnp.uint32)
    val_ref[...] = lax.bitcast_convert_type((hi_u << 16) | lo_u, jnp.float32)

    q = jnp.where(live, q_ref[...], 0.0)
    offy = jnp.dot(q, woffy_ref[...], preferred_element_type=jnp.float32, precision=hi) + boffy_ref[...]
    offx = jnp.dot(q, woffx_ref[...], preferred_element_type=jnp.float32, precision=hi) + boffx_ref[...]
    logits = jnp.dot(q, wattn_ref[...], preferred_element_type=jnp.float32, precision=hi) + battn_ref[...]

    colg = lax.broadcasted_iota(jnp.int32, (HLP, HLP), 0) // (N_LVL * N_PTS)
    colg2 = lax.broadcasted_iota(jnp.int32, (HLP, HLP), 1) // (N_LVL * N_PTS)
    grp = (colg == colg2).astype(jnp.float32)
    e = jnp.exp(logits)
    attn = e / jnp.dot(e, grp, preferred_element_type=jnp.float32, precision=hi)

    rp = jnp.where(live, rp_ref[...], 0.0)
    lcb = (lax.broadcasted_iota(jnp.int32, (1, HLP), 1) % (N_LVL * N_PTS)) // N_PTS
    def _sel(col_for_lvl):
        a0 = rp[:, col_for_lvl(0):col_for_lvl(0) + 1]
        a1 = rp[:, col_for_lvl(1):col_for_lvl(1) + 1]
        a2 = rp[:, col_for_lvl(2):col_for_lvl(2) + 1]
        return jnp.where(lcb == 0, a0, jnp.where(lcb == 1, a1, a2))
    rpy = _sel(lambda l: 2 * l)
    rpx = _sel(lambda l: 2 * l + 1)

    col = lax.broadcasted_iota(jnp.int32, (1, HLP), 1)
    lc = (col % (N_LVL * N_PTS)) // N_PTS
    hcf = (col // (N_LVL * N_PTS)).astype(jnp.float32)
    wf = jnp.where(lc == 0, 100.0, jnp.where(lc == 1, 50.0, 25.0)).astype(jnp.float32)
    lsi = jnp.where(lc == 0, 0.0, jnp.where(lc == 1, 10000.0, 12500.0)).astype(jnp.float32)

    x = rpx * wf + offx - 0.5
    y = rpy * wf + offy - 0.5
    x0 = jnp.floor(x)
    y0 = jnp.floor(y)
    fx = x - x0
    fy = y - y0
    wmax = wf - 1.0
    vx0 = ((x0 >= 0.0) & (x0 <= wmax)).astype(jnp.float32)
    vx1 = ((x0 >= -1.0) & (x0 <= wmax - 1.0)).astype(jnp.float32)
    vy0 = ((y0 >= 0.0) & (y0 <= wmax)).astype(jnp.float32)
    vy1 = ((y0 >= -1.0) & (y0 <= wmax - 1.0)).astype(jnp.float32)
    cx0 = jnp.clip(x0, 0.0, wmax)
    cx1 = jnp.clip(x0 + 1.0, 0.0, wmax)
    cy0 = jnp.clip(y0, 0.0, wmax)
    cy1 = jnp.clip(y0 + 1.0, 0.0, wmax)

    base = bf * jnp.float32(_LQ) + lsi
    gx0 = 1.0 - fx
    gy0 = 1.0 - fy
    corners = (
        (cy0, cx0, gy0 * gx0 * vy0 * vx0),
        (cy0, cx1, gy0 * fx * vy0 * vx1),
        (cy1, cx0, fy * gx0 * vy1 * vx0),
        (cy1, cx1, fy * fx * vy1 * vx1),
    )
    for c, (cy, cx, wbi) in enumerate(corners):
        rowf = (base + cy * wf + cx) * jnp.float32(H_HEADS) + hcf
        idx_ref[:, c, :] = rowf.astype(jnp.int32)
        wgt_ref[:, c, :] = attn * wbi


def _prep_call(q2, vf2, rp2, W_val, b_val, W_off, b_off, W_attn, b_attn):
    woffy = W_off[:, 0::2]
    woffx = W_off[:, 1::2]
    boffy = b_off[0::2].reshape(1, HLP)
    boffx = b_off[1::2].reshape(1, HLP)
    W_val_p = W_val[:, _VAL_PERM]
    b_val_p = b_val[_VAL_PERM]
    row_spec = pl.BlockSpec((_ROW_BLK, D_MODEL), lambda i: (i, 0))
    full = lambda shape: pl.BlockSpec(shape, lambda i: tuple(0 for _ in shape))
    return pl.pallas_call(
        _prep_body,
        grid=(_GRID,),
        in_specs=[
            row_spec,
            row_spec,
            pl.BlockSpec((_ROW_BLK, 2 * N_LVL), lambda i: (i, 0)),
            full((D_MODEL, D_MODEL)),
            full((1, D_MODEL)),
            full((D_MODEL, HLP)),
            full((D_MODEL, HLP)),
            full((1, HLP)),
            full((1, HLP)),
            full((D_MODEL, HLP)),
            full((1, HLP)),
        ],
        out_specs=[
            pl.BlockSpec((_ROW_BLK, _WORDS * H_HEADS), lambda i: (i, 0)),
            pl.BlockSpec((_ROW_BLK, 4, HLP), lambda i: (i, 0, 0)),
            pl.BlockSpec((_ROW_BLK, 4, HLP), lambda i: (i, 0, 0)),
        ],
        out_shape=[
            jax.ShapeDtypeStruct((_NQ_PAD, _WORDS * H_HEADS), jnp.float32),
            jax.ShapeDtypeStruct((_NQ_PAD, 4, HLP), jnp.int32),
            jax.ShapeDtypeStruct((_NQ_PAD, 4, HLP), jnp.float32),
        ],
    )(q2, vf2, rp2, W_val_p, b_val_p.reshape(1, D_MODEL), woffy, woffx,
      boffy, boffx, W_attn, b_attn.reshape(1, HLP))


def _sc_body(table_hbm, idx_hbm, w_hbm, out_hbm,
             idx_a, w_a, rows_a, idx_b, w_b, rows_b, out_v, sem_a, sem_b):
    wid = lax.axis_index("s") * 2 + lax.axis_index("c")
    base_q = wid * _QW

    def fire(c, idx_v, w_v, rows_v, sem):
        q0 = base_q + c * _G
        t0 = pl.multiple_of(q0 * (4 * HLP), 1024)
        r0 = pl.multiple_of(q0 * (4 * HLP) // 128, 8)
        pltpu.sync_copy(idx_hbm.at[pl.ds(r0, _IDX_ROWS)], idx_v)
        pltpu.sync_copy(w_hbm.at[pl.ds(t0, _TERMS)], w_v.at[pl.ds(0, _TERMS)])
        for k in range(_IDX_ROWS):
            pltpu.async_copy(table_hbm.at[idx_v.at[k]],
                             rows_v.at[pl.ds(k * 128, 128)], sem)

    def drain(rows_v, sem):
        pltpu.make_async_copy(table_hbm.at[pl.ds(0, _TERMS)], rows_v, sem).wait()

    mask_hi = jnp.full((16,), 0xFFFF0000, jnp.uint32)

    def compute(c, w_v, rows_v):
        def one_out(o, carry2):
            r = o // H_HEADS
            h = o % H_HEADS
            tb = r * (4 * HLP) + h * (N_LVL * N_PTS)
            acc0 = jnp.zeros((16,), jnp.float32)
            acc1 = jnp.zeros((16,), jnp.float32)
            for cc in range(4):
                w16 = w_v[pl.ds(tb + cc * HLP, 16)]
                for j in range(N_LVL * N_PTS):
                    t = tb + cc * HLP + j
                    wb = w16[j]
                    wu = lax.bitcast_convert_type(rows_v[t, pl.ds(0, _WORDS)],
                                                  jnp.uint32)
                    flo = lax.bitcast_convert_type(wu << 16, jnp.float32)
                    fhi = lax.bitcast_convert_type(wu & mask_hi, jnp.float32)
                    acc0 = acc0 + wb * flo
                    acc1 = acc1 + wb * fhi
            out_v[o, pl.ds(0, 16)] = acc0
            out_v[o, pl.ds(16, 16)] = acc1
            return carry2

        lax.fori_loop(0, _G * H_HEADS, one_out, 0)
        o0 = pl.multiple_of((base_q + c * _G) * H_HEADS, 64)
        pltpu.sync_copy(out_v, out_hbm.at[pl.ds(o0, _G * H_HEADS)])

    fire(0, idx_a, w_a, rows_a, sem_a)

    def pair(p, carry):
        c0 = 2 * p
        fire(c0 + 1, idx_b, w_b, rows_b, sem_b)
        drain(rows_a, sem_a)
        compute(c0, w_a, rows_a)
        fire(c0 + 2, idx_a, w_a, rows_a, sem_a)
        drain(rows_b, sem_b)
        compute(c0 + 1, w_b, rows_b)
        return carry

    lax.fori_loop(0, (_NCH - 1) // 2, pair, 0)
    drain(rows_a, sem_a)
    compute(_NCH - 1, w_a, rows_a)


def _sc_call(table, idx_flat2d, w_flat):
    mesh = plsc.VectorSubcoreMesh(core_axis_name="c", subcore_axis_name="s",
                                  num_cores=2, num_subcores=16)
    f = pl.kernel(
        _sc_body,
        out_type=jax.ShapeDtypeStruct((_NQ_PAD * H_HEADS, HEAD_DIM), jnp.float32),
        mesh=mesh,
        scratch_types=[
            pltpu.VMEM((_IDX_ROWS, 128), jnp.int32),
            pltpu.VMEM((_TERMS + 16,), jnp.float32),
            pltpu.VMEM((_TERMS, _WORDS), jnp.float32),
            pltpu.VMEM((_IDX_ROWS, 128), jnp.int32),
            pltpu.VMEM((_TERMS + 16,), jnp.float32),
            pltpu.VMEM((_TERMS, _WORDS), jnp.float32),
            pltpu.VMEM((_G * H_HEADS, HEAD_DIM), jnp.float32),
            pltpu.SemaphoreType.DMA,
            pltpu.SemaphoreType.DMA,
        ],
        compiler_params=pltpu.CompilerParams(use_tc_tiling_on_sc=False),
    )
    return f(table, idx_flat2d, w_flat)


def _out_body(x_ref, w_ref, b_ref, o_ref):
    o_ref[...] = (
        jnp.dot(x_ref[...], w_ref[...], preferred_element_type=jnp.float32,
                precision=lax.Precision.HIGHEST)
        + b_ref[...])


def _out_call(x2, W_out, b_out):
    row_spec = pl.BlockSpec((_ROW_BLK, D_MODEL), lambda i: (i, 0))
    return pl.pallas_call(
        _out_body,
        grid=(_GRID,),
        in_specs=[
            row_spec,
            pl.BlockSpec((D_MODEL, D_MODEL), lambda i: (0, 0)),
            pl.BlockSpec((1, D_MODEL), lambda i: (0, 0)),
        ],
        out_specs=row_spec,
        out_shape=jax.ShapeDtypeStruct((_NQ_PAD, D_MODEL), jnp.float32),
    )(x2, W_out, b_out.reshape(1, D_MODEL))


def kernel(query, reference_points, value_flat, spatial_shapes, level_start_index,
           W_off, b_off, W_attn, b_attn, W_val, b_val, W_out, b_out):
    B, Lq, d_model = query.shape
    q2 = query.reshape(_NQ, D_MODEL)
    vf2 = value_flat.reshape(_NQ, D_MODEL)
    rp2 = reference_points.reshape(_NQ, 2 * N_LVL)

    val, idx, wgt = _prep_call(q2, vf2, rp2, W_val, b_val, W_off, b_off, W_attn, b_attn)

    table = val.reshape(_NQ_PAD * H_HEADS, _WORDS)
    idx2d = idx.reshape(_NQ_PAD * 4 * HLP // 128, 128)
    wflat = wgt.reshape(_NQ_PAD * 4 * HLP)

    sc_out = _sc_call(table, idx2d, wflat)

    x2 = sc_out.reshape(_NQ_PAD, D_MODEL)
    out = _out_call(x2, W_out, b_out)
    return out[:_NQ].reshape(B, Lq, d_model)


# speedup vs baseline: 84.2316x; 1.0299x over previous
import numpy as np

import jax
import jax.numpy as jnp
from jax import lax
from jax.experimental import pallas as pl
from jax.experimental.pallas import tpu as pltpu
from jax.experimental.pallas import tpu_sc as plsc

H_HEADS = 8
N_LVL = 3
N_PTS = 4
D_MODEL = 256
HEAD_DIM = 32
HLP = H_HEADS * N_LVL * N_PTS

_B = 2
_LQ = 13125
_NQ = _B * _LQ

_NW = 32
_QW = 824
_NQ_PAD = _QW * _NW
_ROW_BLK = 824
_GRID = _NQ_PAD // _ROW_BLK
_G = 8
_NCH = _QW // _G
_TERMS = _G * 4 * HLP
_IDX_ROWS = _TERMS // 128
_WORDS = HEAD_DIM // 2

_VAL_PERM = np.concatenate([
    np.arange(128).reshape(8, 16) // 16 * 32 + np.arange(16)[None, :],
    np.arange(128).reshape(8, 16) // 16 * 32 + 16 + np.arange(16)[None, :],
]).reshape(-1)


def _prep_body(q_ref, vf_ref, rp_ref, wval_ref, bval_ref, woffy_ref, woffx_ref,
               boffy_ref, boffx_ref, wattn_ref, battn_ref, val_ref, idx_ref, wgt_ref):
    i = pl.program_id(0)
    row0 = i * _ROW_BLK
    rowi = row0 + lax.broadcasted_iota(jnp.int32, (_ROW_BLK, 1), 0)
    live = rowi < _NQ
    bf = jnp.minimum(rowi // _LQ, _B - 1).astype(jnp.float32)

    hi = lax.Precision.HIGHEST
    vf = jnp.where(live, vf_ref[...], 0.0)
    val = (jnp.dot(vf, wval_ref[...], preferred_element_type=jnp.float32, precision=hi)
           + bval_ref[...])
    lo_u = lax.bitcast_convert_type(val[:, :128].astype(jnp.bfloat16),
                                    jnp.uint16).astype(jnp.uint32)
    hi_u = lax.bitcast_convert_type(val[:, 128:].astype(jnp.bfloat16),
                                    jnp.uint16).astype(jnp.uint32)
    val_ref[...] = lax.bitcast_convert_type((hi_u << 16) | lo_u, jnp.float32)

    q = jnp.where(live, q_ref[...], 0.0)
    offy = jnp.dot(q, woffy_ref[...], preferred_element_type=jnp.float32, precision=hi) + boffy_ref[...]
    offx = jnp.dot(q, woffx_ref[...], preferred_element_type=jnp.float32, precision=hi) + boffx_ref[...]
    logits = jnp.dot(q, wattn_ref[...], preferred_element_type=jnp.float32, precision=hi) + battn_ref[...]

    colg = lax.broadcasted_iota(jnp.int32, (HLP, HLP), 0) // (N_LVL * N_PTS)
    colg2 = lax.broadcasted_iota(jnp.int32, (HLP, HLP), 1) // (N_LVL * N_PTS)
    grp = (colg == colg2).astype(jnp.float32)
    e = jnp.exp(logits)
    attn = e / jnp.dot(e, grp, preferred_element_type=jnp.float32, precision=hi)

    rp = jnp.where(live, rp_ref[...], 0.0)
    lcb = (lax.broadcasted_iota(jnp.int32, (1, HLP), 1) % (N_LVL * N_PTS)) // N_PTS
    def _sel(col_for_lvl):
        a0 = rp[:, col_for_lvl(0):col_for_lvl(0) + 1]
        a1 = rp[:, col_for_lvl(1):col_for_lvl(1) + 1]
        a2 = rp[:, col_for_lvl(2):col_for_lvl(2) + 1]
        return jnp.where(lcb == 0, a0, jnp.where(lcb == 1, a1, a2))
    rpy = _sel(lambda l: 2 * l)
    rpx = _sel(lambda l: 2 * l + 1)

    col = lax.broadcasted_iota(jnp.int32, (1, HLP), 1)
    lc = (col % (N_LVL * N_PTS)) // N_PTS
    hcf = (col // (N_LVL * N_PTS)).astype(jnp.float32)
    wf = jnp.where(lc == 0, 100.0, jnp.where(lc == 1, 50.0, 25.0)).astype(jnp.float32)
    lsi = jnp.where(lc == 0, 0.0, jnp.where(lc == 1, 10000.0, 12500.0)).astype(jnp.float32)

    x = rpx * wf + offx - 0.5
    y = rpy * wf + offy - 0.5
    x0 = jnp.floor(x)
    y0 = jnp.floor(y)
    fx = x - x0
    fy = y - y0
    wmax = wf - 1.0
    vx0 = ((x0 >= 0.0) & (x0 <= wmax)).astype(jnp.float32)
    vx1 = ((x0 >= -1.0) & (x0 <= wmax - 1.0)).astype(jnp.float32)
    vy0 = ((y0 >= 0.0) & (y0 <= wmax)).astype(jnp.float32)
    vy1 = ((y0 >= -1.0) & (y0 <= wmax - 1.0)).astype(jnp.float32)
    cx0 = jnp.clip(x0, 0.0, wmax)
    cx1 = jnp.clip(x0 + 1.0, 0.0, wmax)
    cy0 = jnp.clip(y0, 0.0, wmax)
    cy1 = jnp.clip(y0 + 1.0, 0.0, wmax)

    base = bf * jnp.float32(_LQ) + lsi
    gx0 = 1.0 - fx
    gy0 = 1.0 - fy
    corners = (
        (cy0, cx0, gy0 * gx0 * vy0 * vx0),
        (cy0, cx1, gy0 * fx * vy0 * vx1),
        (cy1, cx0, fy * gx0 * vy1 * vx0),
        (cy1, cx1, fy * fx * vy1 * vx1),
    )
    for c, (cy, cx, wbi) in enumerate(corners):
        rowf = (base + cy * wf + cx) * jnp.float32(H_HEADS) + hcf
        idx_ref[:, c, :] = rowf.astype(jnp.int32)
        wgt_ref[:, c, :] = attn * wbi


def _prep_call(q2, vf2, rp2, W_val, b_val, W_off, b_off, W_attn, b_attn):
    woffy = W_off[:, 0::2]
    woffx = W_off[:, 1::2]
    boffy = b_off[0::2].reshape(1, HLP)
    boffx = b_off[1::2].reshape(1, HLP)
    W_val_p = W_val[:, _VAL_PERM]
    b_val_p = b_val[_VAL_PERM]
    row_spec = pl.BlockSpec((_ROW_BLK, D_MODEL), lambda i: (i, 0))
    full = lambda shape: pl.BlockSpec(shape, lambda i: tuple(0 for _ in shape))
    return pl.pallas_call(
        _prep_body,
        grid=(_GRID,),
        in_specs=[
            row_spec,
            row_spec,
            pl.BlockSpec((_ROW_BLK, 2 * N_LVL), lambda i: (i, 0)),
            full((D_MODEL, D_MODEL)),
            full((1, D_MODEL)),
            full((D_MODEL, HLP)),
            full((D_MODEL, HLP)),
            full((1, HLP)),
            full((1, HLP)),
            full((D_MODEL, HLP)),
            full((1, HLP)),
        ],
        out_specs=[
            pl.BlockSpec((_ROW_BLK, _WORDS * H_HEADS), lambda i: (i, 0)),
            pl.BlockSpec((_ROW_BLK, 4, HLP), lambda i: (i, 0, 0)),
            pl.BlockSpec((_ROW_BLK, 4, HLP), lambda i: (i, 0, 0)),
        ],
        out_shape=[
            jax.ShapeDtypeStruct((_NQ_PAD, _WORDS * H_HEADS), jnp.float32),
            jax.ShapeDtypeStruct((_NQ_PAD, 4, HLP), jnp.int32),
            jax.ShapeDtypeStruct((_NQ_PAD, 4, HLP), jnp.float32),
        ],
    )(q2, vf2, rp2, W_val_p, b_val_p.reshape(1, D_MODEL), woffy, woffx,
      boffy, boffx, W_attn, b_attn.reshape(1, HLP))


def _sc_body(table_hbm, idx_hbm, w_hbm, out_hbm,
             idx_a, w_a, rows_a, idx_b, w_b, rows_b, out_v, sem_a, sem_b):
    wid = lax.axis_index("s") * 2 + lax.axis_index("c")
    base_q = wid * _QW

    def fire(c, idx_v, w_v, rows_v, sem):
        q0 = base_q + c * _G
        t0 = pl.multiple_of(q0 * (4 * HLP), 1024)
        r0 = pl.multiple_of(q0 * (4 * HLP) // 128, 8)
        pltpu.sync_copy(idx_hbm.at[pl.ds(r0, _IDX_ROWS)], idx_v)
        pltpu.sync_copy(w_hbm.at[pl.ds(t0, _TERMS)], w_v.at[pl.ds(0, _TERMS)])
        for k in range(_IDX_ROWS):
            pltpu.async_copy(table_hbm.at[idx_v.at[k]],
                             rows_v.at[pl.ds(k * 128, 128)], sem)

    def drain(rows_v, sem):
        pltpu.make_async_copy(table_hbm.at[pl.ds(0, _TERMS)], rows_v, sem).wait()

    mask_hi = jnp.full((16,), 0xFFFF0000, jnp.uint32)

    def compute(c, w_v, rows_v):
        def one_out(o, carry2):
            r = o // H_HEADS
            h = o % H_HEADS
            tb = r * (4 * HLP) + h * (N_LVL * N_PTS)
            acc0 = jnp.zeros((16,), jnp.float32)
            acc1 = jnp.zeros((16,), jnp.float32)
            for cc in range(4):
                w16 = w_v[pl.ds(tb + cc * HLP, 16)]
                for j in range(N_LVL * N_PTS):
                    t = tb + cc * HLP + j
                    wb = w16[j]
                    wu = lax.bitcast_convert_type(rows_v[t, pl.ds(0, _WORDS)],
                                                  jnp.uint32)
                    flo = lax.bitcast_convert_type(wu << 16, jnp.float32)
                    fhi = lax.bitcast_convert_type(wu & mask_hi, jnp.float32)
                    acc0 = acc0 + wb * flo
                    acc1 = acc1 + wb * fhi
            out_v[r, pl.ds(h * HEAD_DIM, 16)] = acc0
            out_v[r, pl.ds(h * HEAD_DIM + 16, 16)] = acc1
            return carry2

        lax.fori_loop(0, _G * H_HEADS, one_out, 0)
        o0 = pl.multiple_of(base_q + c * _G, 8)
        pltpu.sync_copy(out_v, out_hbm.at[pl.ds(o0, _G)])

    fire(0, idx_a, w_a, rows_a, sem_a)

    def pair(p, carry):
        c0 = 2 * p
        fire(c0 + 1, idx_b, w_b, rows_b, sem_b)
        drain(rows_a, sem_a)
        compute(c0, w_a, rows_a)
        fire(c0 + 2, idx_a, w_a, rows_a, sem_a)
        drain(rows_b, sem_b)
        compute(c0 + 1, w_b, rows_b)
        return carry

    lax.fori_loop(0, (_NCH - 1) // 2, pair, 0)
    drain(rows_a, sem_a)
    compute(_NCH - 1, w_a, rows_a)


def _sc_call(table, idx_flat2d, w_flat):
    mesh = plsc.VectorSubcoreMesh(core_axis_name="c", subcore_axis_name="s",
                                  num_cores=2, num_subcores=16)
    f = pl.kernel(
        _sc_body,
        out_type=jax.ShapeDtypeStruct((_NQ_PAD, D_MODEL), jnp.float32),
        mesh=mesh,
        scratch_types=[
            pltpu.VMEM((_IDX_ROWS, 128), jnp.int32),
            pltpu.VMEM((_TERMS + 16,), jnp.float32),
            pltpu.VMEM((_TERMS, _WORDS), jnp.float32),
            pltpu.VMEM((_IDX_ROWS, 128), jnp.int32),
            pltpu.VMEM((_TERMS + 16,), jnp.float32),
            pltpu.VMEM((_TERMS, _WORDS), jnp.float32),
            pltpu.VMEM((_G, D_MODEL), jnp.float32),
            pltpu.SemaphoreType.DMA,
            pltpu.SemaphoreType.DMA,
        ],
        compiler_params=pltpu.CompilerParams(use_tc_tiling_on_sc=False),
    )
    return f(table, idx_flat2d, w_flat)


def _out_body(x_ref, w_ref, b_ref, o_ref):
    o_ref[...] = (
        jnp.dot(x_ref[...], w_ref[...], preferred_element_type=jnp.float32,
                precision=lax.Precision.HIGHEST)
        + b_ref[...])


def _out_call(x2, W_out, b_out):
    row_spec = pl.BlockSpec((_ROW_BLK, D_MODEL), lambda i: (i, 0))
    return pl.pallas_call(
        _out_body,
        grid=(_GRID,),
        in_specs=[
            row_spec,
            pl.BlockSpec((D_MODEL, D_MODEL), lambda i: (0, 0)),
            pl.BlockSpec((1, D_MODEL), lambda i: (0, 0)),
        ],
        out_specs=row_spec,
        out_shape=jax.ShapeDtypeStruct((_NQ, D_MODEL), jnp.float32),
    )(x2, W_out, b_out.reshape(1, D_MODEL))


def kernel(query, reference_points, value_flat, spatial_shapes, level_start_index,
           W_off, b_off, W_attn, b_attn, W_val, b_val, W_out, b_out):
    B, Lq, d_model = query.shape
    q2 = query.reshape(_NQ, D_MODEL)
    vf2 = value_flat.reshape(_NQ, D_MODEL)
    rp2 = reference_points.reshape(_NQ, 2 * N_LVL)

    val, idx, wgt = _prep_call(q2, vf2, rp2, W_val, b_val, W_off, b_off, W_attn, b_attn)

    table = val.reshape(_NQ_PAD * H_HEADS, _WORDS)
    idx2d = idx.reshape(_NQ_PAD * 4 * HLP // 128, 128)
    wflat = wgt.reshape(_NQ_PAD * 4 * HLP)

    sc_out = _sc_call(table, idx2d, wflat)

    out = _out_call(sc_out, W_out, b_out)
    return out.reshape(B, Lq, d_model)
